# Initial kernel scaffold; baseline (speedup 1.0000x reference)
import jax
import jax.numpy as jnp
from jax.experimental import pallas as pl


def kernel(query, key, value, Wq, bq, Wk, bk, Wv, bv, Wo, bo, hash_w):
    raise NotImplementedError("write your pallas kernel here")


# speedup vs baseline: 3.2856x; 3.2856x over previous
import jax
import jax.numpy as jnp
from jax import lax
from jax.experimental import pallas as pl

T, B, E, H = 4096, 1, 1024, 16
R, NH, CS = 2, 8, 64
Dh = E // H
SCALING = Dh ** -0.5
NC = T // CS
NP = R * H

_INTERPRET = False

_bf16 = jnp.bfloat16
_f32 = jnp.float32


def _dot_bf16(a, b):
    return lax.dot(a.astype(_bf16), b.astype(_bf16),
                   preferred_element_type=_f32)


_TBA = 512


def _proj_hash_body(xq_ref, xv_ref, wqt_ref, bq_ref, wvt_ref, bv_ref,
                    wh_ref, qsc_ref, vp_ref, invs_ref, hash_ref):
    q = _dot_bf16(xq_ref[...], wqt_ref[...]) + bq_ref[...]
    norm = jnp.sqrt(jnp.sum(q * q, axis=1, keepdims=True))
    kh = q / norm
    v = _dot_bf16(xv_ref[...], wvt_ref[...]) + bv_ref[...]
    lin = _dot_bf16(kh, wh_ref[...])
    l0 = lin[:, 0:32]
    babs = jnp.abs(l0)
    bidx = jnp.zeros(l0.shape, jnp.int32)
    bval = l0
    for c in range(1, 4):
        lc = lin[:, c * 32:(c + 1) * 32]
        m = jnp.abs(lc) > babs
        babs = jnp.where(m, jnp.abs(lc), babs)
        bidx = jnp.where(m, c, bidx)
        bval = jnp.where(m, lc, bval)
    hsh = bidx + 4 * (bval < 0).astype(jnp.int32)
    qsc_ref[...] = q * SCALING
    vp_ref[...] = v
    invs_ref[...] = 1.0 / (norm * SCALING)
    hash_ref[...] = hsh


def _stage_a(xq, xv, wqt, bq2, wvt, bv2, wh):
    return pl.pallas_call(
        _proj_hash_body,
        grid=(T // _TBA,),
        in_specs=[
            pl.BlockSpec((_TBA, E), lambda i: (i, 0)),
            pl.BlockSpec((_TBA, E), lambda i: (i, 0)),
            pl.BlockSpec((E, E), lambda i: (0, 0)),
            pl.BlockSpec((1, E), lambda i: (0, 0)),
            pl.BlockSpec((E, E), lambda i: (0, 0)),
            pl.BlockSpec((1, E), lambda i: (0, 0)),
            pl.BlockSpec((E, 4 * NP), lambda i: (0, 0)),
        ],
        out_specs=[
            pl.BlockSpec((_TBA, E), lambda i: (i, 0)),
            pl.BlockSpec((_TBA, E), lambda i: (i, 0)),
            pl.BlockSpec((_TBA, 1), lambda i: (i, 0)),
            pl.BlockSpec((_TBA, NP), lambda i: (i, 0)),
        ],
        out_shape=[
            jax.ShapeDtypeStruct((T, E), _f32),
            jax.ShapeDtypeStruct((T, E), _f32),
            jax.ShapeDtypeStruct((T, 1), _f32),
            jax.ShapeDtypeStruct((T, NP), jnp.int32),
        ],
        interpret=_INTERPRET,
    )(xq, xv, wqt, bq2, wvt, bv2, wh)


def _attn_body(sq_ref, sv_ref, sinvc_ref, shc_ref, spc_ref, shr_ref,
               spr_ref, out_ref, lse_ref):
    sq = sq_ref[0]
    ks = sq * sinvc_ref[0]
    sv = sv_ref[0]
    hcol = shc_ref[0]
    pcol = spc_ref[0]
    hrow = shr_ref[0]
    prow = spr_ref[0]

    for n in range(NC):
        a = n * CS
        am = (a + T - CS) % T
        ap = (a + CS) % T
        qn = sq[a:a + CS, :]
        hq = hcol[a:a + CS, :]
        pq = pcol[a:a + CS, :]
        kw = jnp.concatenate(
            [ks[am:am + CS, :], ks[a:a + CS, :], ks[ap:ap + CS, :]], axis=0)
        vw = jnp.concatenate(
            [sv[am:am + CS, :], sv[a:a + CS, :], sv[ap:ap + CS, :]], axis=0)
        hw = jnp.concatenate(
            [hrow[:, am:am + CS], hrow[:, a:a + CS], hrow[:, ap:ap + CS]],
            axis=1)
        pw = jnp.concatenate(
            [prow[:, am:am + CS], prow[:, a:a + CS], prow[:, ap:ap + CS]],
            axis=1)
        s = lax.dot_general(qn.astype(_bf16), kw.astype(_bf16),
                            (((1,), (1,)), ((), ())),
                            preferred_element_type=_f32)
        s = jnp.where(hq != hw, -1e9, s)
        s = jnp.where(pq == pw, -1e5, s)
        m = jnp.max(s, axis=1, keepdims=True)
        lse = m + jnp.log(jnp.sum(jnp.exp(s - m), axis=1, keepdims=True))
        probs = jnp.exp(s - lse)
        o = lax.dot_general(probs.astype(_bf16), vw.astype(_bf16),
                            (((1,), (0,)), ((), ())),
                            preferred_element_type=_f32)
        out_ref[0, a:a + CS, :] = o
        lse_ref[0, a:a + CS, :] = lse


def _stage_e(sq, sv, sinvc, shc, spc, shr, spr):
    return pl.pallas_call(
        _attn_body,
        grid=(NP,),
        in_specs=[
            pl.BlockSpec((1, T, Dh), lambda i: (i, 0, 0)),
            pl.BlockSpec((1, T, Dh), lambda i: (i, 0, 0)),
            pl.BlockSpec((1, T, 1), lambda i: (i, 0, 0)),
            pl.BlockSpec((1, T, 1), lambda i: (i, 0, 0)),
            pl.BlockSpec((1, T, 1), lambda i: (i, 0, 0)),
            pl.BlockSpec((1, 1, T), lambda i: (i, 0, 0)),
            pl.BlockSpec((1, 1, T), lambda i: (i, 0, 0)),
        ],
        out_specs=[
            pl.BlockSpec((1, T, Dh), lambda i: (i, 0, 0)),
            pl.BlockSpec((1, T, 1), lambda i: (i, 0, 0)),
        ],
        out_shape=[
            jax.ShapeDtypeStruct((NP, T, Dh), _f32),
            jax.ShapeDtypeStruct((NP, T, 1), _f32),
        ],
        interpret=_INTERPRET,
    )(sq, sv, sinvc, shc, spc, shr, spr)


_TBD = 512


def _combine_body(ou_ref, lse_ref, b16_ref, wot_ref, bo_ref, out_ref):
    ou = ou_ref[...]
    o0 = ou[:, :E]
    o1 = ou[:, E:]
    l = lse_ref[...]
    d = l[0:H, :] - l[H:NP, :]
    w0 = jax.nn.sigmoid(d)
    w1 = jax.nn.sigmoid(-d)
    b16 = b16_ref[...]
    w0f = lax.dot_general(w0, b16, (((0,), (0,)), ((), ())),
                          precision=lax.Precision.HIGHEST,
                          preferred_element_type=_f32)
    w1f = lax.dot_general(w1, b16, (((0,), (0,)), ((), ())),
                          precision=lax.Precision.HIGHEST,
                          preferred_element_type=_f32)
    comb = o0 * w0f + o1 * w1f
    out_ref[...] = _dot_bf16(comb, wot_ref[...]) + bo_ref[...]


def _stage_d(outu, lseu, b16, wot, bo2):
    return pl.pallas_call(
        _combine_body,
        grid=(T // _TBD,),
        in_specs=[
            pl.BlockSpec((_TBD, 2 * E), lambda i: (i, 0)),
            pl.BlockSpec((NP, _TBD), lambda i: (0, i)),
            pl.BlockSpec((H, E), lambda i: (0, 0)),
            pl.BlockSpec((E, E), lambda i: (0, 0)),
            pl.BlockSpec((1, E), lambda i: (0, 0)),
        ],
        out_specs=pl.BlockSpec((_TBD, E), lambda i: (i, 0)),
        out_shape=jax.ShapeDtypeStruct((T, E), _f32),
        interpret=_INTERPRET,
    )(outu, lseu, b16, wot, bo2)


def kernel(query, key, value, Wq, bq, Wk, bk, Wv, bv, Wo, bo, hash_w):
    xq = query.reshape(T, E)
    xv = value.reshape(T, E)
    wqt = Wq.T
    wvt = Wv.T
    wot = Wo.T
    bq2 = bq.reshape(1, E)
    bv2 = bv.reshape(1, E)
    bo2 = bo.reshape(1, E)
    wh = jnp.einsum('rhfc,hg->hfcrg', hash_w,
                    jnp.eye(H, dtype=_f32)).reshape(E, 4 * NP)
    b16 = jnp.repeat(jnp.eye(H, dtype=_f32), Dh, axis=1)

    qsc, vp, invs, hashes = _stage_a(xq, xv, wqt, bq2, wvt, bv2, wh)

    hashes_t = hashes.T
    sidx = jnp.argsort(hashes_t, axis=-1, stable=True).astype(jnp.int32)
    shash = jnp.take_along_axis(hashes_t, sidx, axis=-1)
    sinvs = invs.reshape(T)[sidx]
    qh = qsc.reshape(T, H, Dh).transpose(1, 0, 2)
    vh = vp.reshape(T, H, Dh).transpose(1, 0, 2)
    hmap = jnp.arange(NP) % H
    sq = jnp.take_along_axis(qh[hmap], sidx[:, :, None], axis=1)
    sv = jnp.take_along_axis(vh[hmap], sidx[:, :, None], axis=1)

    outs, lses = _stage_e(sq, sv, sinvs[:, :, None], shash[:, :, None],
                          sidx[:, :, None], shash[:, None, :],
                          sidx[:, None, :])

    inv = jnp.argsort(sidx, axis=-1)
    outu32 = jnp.take_along_axis(outs, inv[:, :, None], axis=1)
    lseu = jnp.take_along_axis(lses[:, :, 0], inv, axis=1)
    outu = outu32.reshape(R, H, T, Dh).transpose(2, 0, 1, 3).reshape(T, 2 * E)

    out = _stage_d(outu, lseu, b16, wot, bo2)
    return out.reshape(T, B, E)


# speedup vs baseline: 6.3138x; 1.9217x over previous
import functools

import jax
import jax.numpy as jnp
from jax import lax
from jax.experimental import pallas as pl
from jax.experimental.pallas import tpu as pltpu
from jax.experimental.pallas import tpu_sc as plsc

T, B, E, H = 4096, 1, 1024, 16
R, NH, CS = 2, 8, 64
Dh = E // H
SCALING = Dh ** -0.5
NC = T // CS
NP = R * H

_INTERPRET = False

_bf16 = jnp.bfloat16
_f32 = jnp.float32


def _dot_bf16(a, b):
    return lax.dot(a.astype(_bf16), b.astype(_bf16),
                   preferred_element_type=_f32)


_TBA = 512


def _proj_hash_body(xq_ref, xv_ref, wqt_ref, bq_ref, wvt_ref, bv_ref,
                    wh_ref, qsc_ref, vp_ref, invs_ref, hash_ref):
    q = _dot_bf16(xq_ref[...], wqt_ref[...]) + bq_ref[...]
    norm = jnp.sqrt(jnp.sum(q * q, axis=1, keepdims=True))
    kh = q / norm
    v = _dot_bf16(xv_ref[...], wvt_ref[...]) + bv_ref[...]
    lin = _dot_bf16(kh, wh_ref[...])
    l0 = lin[:, 0:32]
    babs = jnp.abs(l0)
    bidx = jnp.zeros(l0.shape, jnp.int32)
    bval = l0
    for c in range(1, 4):
        lc = lin[:, c * 32:(c + 1) * 32]
        m = jnp.abs(lc) > babs
        babs = jnp.where(m, jnp.abs(lc), babs)
        bidx = jnp.where(m, c, bidx)
        bval = jnp.where(m, lc, bval)
    hsh = bidx + 4 * (bval < 0).astype(jnp.int32)
    qsc_ref[...] = q * SCALING
    vp_ref[...] = v
    invs_ref[...] = 1.0 / (norm * SCALING)
    hash_ref[...] = hsh


def _stage_a(xq, xv, wqt, bq2, wvt, bv2, wh):
    return pl.pallas_call(
        _proj_hash_body,
        grid=(T // _TBA,),
        in_specs=[
            pl.BlockSpec((_TBA, E), lambda i: (i, 0)),
            pl.BlockSpec((_TBA, E), lambda i: (i, 0)),
            pl.BlockSpec((E, E), lambda i: (0, 0)),
            pl.BlockSpec((1, E), lambda i: (0, 0)),
            pl.BlockSpec((E, E), lambda i: (0, 0)),
            pl.BlockSpec((1, E), lambda i: (0, 0)),
            pl.BlockSpec((E, 4 * NP), lambda i: (0, 0)),
        ],
        out_specs=[
            pl.BlockSpec((_TBA, E), lambda i: (i, 0)),
            pl.BlockSpec((_TBA, E), lambda i: (i, 0)),
            pl.BlockSpec((_TBA, 1), lambda i: (i, 0)),
            pl.BlockSpec((_TBA, NP), lambda i: (i, 0)),
        ],
        out_shape=[
            jax.ShapeDtypeStruct((T, E), _f32),
            jax.ShapeDtypeStruct((T, E), _f32),
            jax.ShapeDtypeStruct((T, 1), _f32),
            jax.ShapeDtypeStruct((T, NP), jnp.int32),
        ],
        interpret=_INTERPRET,
    )(xq, xv, wqt, bq2, wvt, bv2, wh)


_GC = 128
_NGC = T // _GC
_NV = T // 16


def _sc_wid():
    return lax.axis_index("s") * 2 + lax.axis_index("c")


def _sort_gather_body(hash_hbm, invs_hbm, qv_hbm,
                      sidx_hbm, shash_hbm, sinvs_hbm, sqv_hbm,
                      hv, invs_v, sidx_v, shash_v, sinvs_v, idx2, rowbuf,
                      nxt, sem):
    wid = _sc_wid()
    hh = lax.rem(wid, H)
    pltpu.sync_copy(hash_hbm.at[wid], hv)
    pltpu.sync_copy(invs_hbm, invs_v)

    def h1(i, carry):
        hvec = hv[pl.ds(i * 16, 16)]
        return tuple(carry[b] + jnp.sum((hvec == b).astype(jnp.int32))
                     for b in range(NH))

    c8 = lax.fori_loop(0, _NV, h1, (jnp.int32(0),) * NH)
    off = jnp.int32(0)
    for b in range(NH):
        nxt[b] = off
        off = off + c8[b]

    def h2(i, carry):
        hvec = hv[pl.ds(i * 16, 16)]
        pos = jnp.zeros((16,), jnp.int32)
        for b in range(NH):
            eq = hvec == b
            eqi = eq.astype(jnp.int32)
            cs = plsc.cumsum(eqi)
            nb = nxt[b]
            nbv = jnp.full((16,), nb, jnp.int32)
            pos = jnp.where(eq, nbv + cs - 1, pos)
            nxt[b] = nb + jnp.sum(eqi)
        tok = jnp.full((16,), i * 16, jnp.int32) + lax.iota(jnp.int32, 16)
        plsc.store_scatter(sidx_v, [pos], tok)
        plsc.store_scatter(shash_v, [pos], hvec)
        return carry

    lax.fori_loop(0, _NV, h2, 0)

    for c in range(_NGC):
        def h3(j, carry):
            base = c * _GC + j * 16
            iv = sidx_v[pl.ds(base, 16)]
            sinvs_v[pl.ds(base, 16)] = plsc.load_gather(invs_v, [iv])
            idx2[c, pl.ds(j * 16, 16)] = iv * H + jnp.full((16,), hh,
                                                          jnp.int32)
            return carry
        lax.fori_loop(0, _GC // 16, h3, 0)

    pltpu.sync_copy(sidx_v, sidx_hbm.at[wid])
    pltpu.sync_copy(shash_v, shash_hbm.at[wid])
    pltpu.sync_copy(sinvs_v, sinvs_hbm.at[wid])

    def gq(c, carry):
        pltpu.async_copy(qv_hbm.at[idx2.at[c]], rowbuf, sem).wait()
        pltpu.sync_copy(rowbuf, sqv_hbm.at[wid, pl.ds(c * _GC, _GC)])
        return carry

    lax.fori_loop(0, _NGC, gq, 0)


def _stage_b(hashes_t, invs_flat, qv):
    mesh = plsc.VectorSubcoreMesh(core_axis_name="c", subcore_axis_name="s")
    f = functools.partial(
        pl.kernel, mesh=mesh,
        out_type=[
            jax.ShapeDtypeStruct((NP, T), jnp.int32),
            jax.ShapeDtypeStruct((NP, T), jnp.int32),
            jax.ShapeDtypeStruct((NP, T), _f32),
            jax.ShapeDtypeStruct((NP, T, 2 * Dh), _f32),
        ],
        scratch_types=[
            pltpu.VMEM((T,), jnp.int32),
            pltpu.VMEM((T,), _f32),
            pltpu.VMEM((T,), jnp.int32),
            pltpu.VMEM((T,), jnp.int32),
            pltpu.VMEM((T,), _f32),
            pltpu.VMEM((_NGC, _GC), jnp.int32),
            pltpu.VMEM((_GC, 2 * Dh), _f32),
            pltpu.SMEM((16,), jnp.int32),
            pltpu.SemaphoreType.DMA,
        ],
        compiler_params=pltpu.CompilerParams(needs_layout_passes=False),
    )(_sort_gather_body)
    return f(hashes_t, invs_flat, qv)


def _unsort_body(outp_hbm, sidx_hbm, outu_hbm,
                 sidx_v, idx3, rowbuf, sem):
    wid = _sc_wid()
    pltpu.sync_copy(sidx_hbm.at[wid], sidx_v)

    for c in range(_NGC):
        def f1(j, carry):
            base = c * _GC + j * 16
            iv = sidx_v[pl.ds(base, 16)]
            idx3[c, pl.ds(j * 16, 16)] = iv * NP + jnp.full((16,), wid,
                                                            jnp.int32)
            return carry
        lax.fori_loop(0, _GC // 16, f1, 0)

    def f2(c, carry):
        pltpu.sync_copy(outp_hbm.at[wid, pl.ds(c * _GC, _GC)], rowbuf)
        pltpu.async_copy(rowbuf, outu_hbm.at[idx3.at[c]], sem).wait()
        return carry

    lax.fori_loop(0, _NGC, f2, 0)


def _stage_c(outp, sidx):
    mesh = plsc.VectorSubcoreMesh(core_axis_name="c", subcore_axis_name="s")
    f = functools.partial(
        pl.kernel, mesh=mesh,
        out_type=[
            jax.ShapeDtypeStruct((T * NP, 2 * Dh), _f32),
        ],
        scratch_types=[
            pltpu.VMEM((T,), jnp.int32),
            pltpu.VMEM((_NGC, _GC), jnp.int32),
            pltpu.VMEM((_GC, 2 * Dh), _f32),
            pltpu.SemaphoreType.DMA,
        ],
        compiler_params=pltpu.CompilerParams(needs_layout_passes=False),
    )(_unsort_body)
    return f(outp, sidx)[0]


def _attn_body(sqv_ref, sinvc_ref, shc_ref, spc_ref, shr_ref,
               spr_ref, out_ref):
    sqv = sqv_ref[0]
    sq = sqv[:, 0:Dh]
    sv = sqv[:, Dh:2 * Dh]
    ks = sq * sinvc_ref[0]
    hcol = shc_ref[0]
    pcol = spc_ref[0]
    hrow = shr_ref[0]
    prow = spr_ref[0]

    for n in range(NC):
        a = n * CS
        am = (a + T - CS) % T
        ap = (a + CS) % T
        qn = sq[a:a + CS, :]
        hq = hcol[a:a + CS, :]
        pq = pcol[a:a + CS, :]
        kw = jnp.concatenate(
            [ks[am:am + CS, :], ks[a:a + CS, :], ks[ap:ap + CS, :]], axis=0)
        vw = jnp.concatenate(
            [sv[am:am + CS, :], sv[a:a + CS, :], sv[ap:ap + CS, :]], axis=0)
        hw = jnp.concatenate(
            [hrow[:, am:am + CS], hrow[:, a:a + CS], hrow[:, ap:ap + CS]],
            axis=1)
        pw = jnp.concatenate(
            [prow[:, am:am + CS], prow[:, a:a + CS], prow[:, ap:ap + CS]],
            axis=1)
        s = lax.dot_general(qn.astype(_bf16), kw.astype(_bf16),
                            (((1,), (1,)), ((), ())),
                            preferred_element_type=_f32)
        s = jnp.where(hq != hw, -1e9, s)
        s = jnp.where(pq == pw, -1e5, s)
        m = jnp.max(s, axis=1, keepdims=True)
        lse = m + jnp.log(jnp.sum(jnp.exp(s - m), axis=1, keepdims=True))
        probs = jnp.exp(s - lse)
        o = lax.dot_general(probs.astype(_bf16), vw.astype(_bf16),
                            (((1,), (0,)), ((), ())),
                            preferred_element_type=_f32)
        out_ref[0, a:a + CS, 0:Dh] = o
        out_ref[0, a:a + CS, Dh:Dh + 1] = lse


def _stage_e(sqv, sinvc, shc, spc, shr, spr):
    return pl.pallas_call(
        _attn_body,
        grid=(NP,),
        in_specs=[
            pl.BlockSpec((1, T, 2 * Dh), lambda i: (i, 0, 0)),
            pl.BlockSpec((1, T, 1), lambda i: (i, 0, 0)),
            pl.BlockSpec((1, T, 1), lambda i: (i, 0, 0)),
            pl.BlockSpec((1, T, 1), lambda i: (i, 0, 0)),
            pl.BlockSpec((1, 1, T), lambda i: (i, 0, 0)),
            pl.BlockSpec((1, 1, T), lambda i: (i, 0, 0)),
        ],
        out_specs=pl.BlockSpec((1, T, 2 * Dh), lambda i: (i, 0, 0)),
        out_shape=jax.ShapeDtypeStruct((NP, T, 2 * Dh), _f32),
        interpret=_INTERPRET,
    )(sqv, sinvc, shc, spc, shr, spr)


_TBD = 512


def _combine_body(ou_ref, wot_ref, bo_ref, out_ref):
    pieces = []
    for h in range(H):
        c0 = h * 2 * Dh
        c1 = (H + h) * 2 * Dh
        o0 = ou_ref[:, c0:c0 + Dh]
        l0 = ou_ref[:, c0 + Dh:c0 + Dh + 1]
        o1 = ou_ref[:, c1:c1 + Dh]
        l1 = ou_ref[:, c1 + Dh:c1 + Dh + 1]
        d = l0 - l1
        w0 = jax.nn.sigmoid(d)
        w1 = jax.nn.sigmoid(-d)
        pieces.append(o0 * w0 + o1 * w1)
    comb = jnp.concatenate(pieces, axis=1)
    out_ref[...] = _dot_bf16(comb, wot_ref[...]) + bo_ref[...]


def _stage_d(outu, wot, bo2):
    return pl.pallas_call(
        _combine_body,
        grid=(T // _TBD,),
        in_specs=[
            pl.BlockSpec((_TBD, NP * 2 * Dh), lambda i: (i, 0)),
            pl.BlockSpec((E, E), lambda i: (0, 0)),
            pl.BlockSpec((1, E), lambda i: (0, 0)),
        ],
        out_specs=pl.BlockSpec((_TBD, E), lambda i: (i, 0)),
        out_shape=jax.ShapeDtypeStruct((T, E), _f32),
        interpret=_INTERPRET,
    )(outu, wot, bo2)


def kernel(query, key, value, Wq, bq, Wk, bk, Wv, bv, Wo, bo, hash_w):
    xq = query.reshape(T, E)
    xv = value.reshape(T, E)
    wqt = Wq.T
    wvt = Wv.T
    wot = Wo.T
    bq2 = bq.reshape(1, E)
    bv2 = bv.reshape(1, E)
    bo2 = bo.reshape(1, E)
    wh = jnp.einsum('rhfc,hg->hfcrg', hash_w,
                    jnp.eye(H, dtype=_f32)).reshape(E, 4 * NP)

    qsc, vp, invs, hashes = _stage_a(xq, xv, wqt, bq2, wvt, bv2, wh)

    hashes_t = hashes.T
    qv = jnp.concatenate([qsc.reshape(T, H, Dh), vp.reshape(T, H, Dh)],
                         axis=-1).reshape(T * H, 2 * Dh)
    sidx, shash, sinvs, sqv = _stage_b(hashes_t, invs.reshape(T), qv)

    outp = _stage_e(sqv, sinvs[:, :, None], shash[:, :, None],
                    sidx[:, :, None], shash[:, None, :], sidx[:, None, :])

    outu_flat = _stage_c(outp, sidx)
    outu = outu_flat.reshape(T, NP * 2 * Dh)

    out = _stage_d(outu, wot, bo2)
    return out.reshape(T, B, E)


# speedup vs baseline: 8.0261x; 1.2712x over previous
import functools

import jax
import jax.numpy as jnp
from jax import lax
from jax.experimental import pallas as pl
from jax.experimental.pallas import tpu as pltpu
from jax.experimental.pallas import tpu_sc as plsc

T, B, E, H = 4096, 1, 1024, 16
R, NH, CS = 2, 8, 64
Dh = E // H
SCALING = Dh ** -0.5
NC = T // CS
NP = R * H

_INTERPRET = False

_bf16 = jnp.bfloat16
_f32 = jnp.float32


def _dot_bf16(a, b):
    return lax.dot(a.astype(_bf16), b.astype(_bf16),
                   preferred_element_type=_f32)


_TBA = 512


def _proj_hash_body(xq_ref, xv_ref, wqt_ref, bq_ref, wvt_ref, bv_ref,
                    wh_ref, qsc_ref, vp_ref, invs_ref, hash_ref):
    q = _dot_bf16(xq_ref[...], wqt_ref[...]) + bq_ref[...]
    norm = jnp.sqrt(jnp.sum(q * q, axis=1, keepdims=True))
    kh = q / norm
    v = _dot_bf16(xv_ref[...], wvt_ref[...]) + bv_ref[...]
    lin = _dot_bf16(kh, wh_ref[...])
    l0 = lin[:, 0:32]
    babs = jnp.abs(l0)
    bidx = jnp.zeros(l0.shape, jnp.int32)
    bval = l0
    for c in range(1, 4):
        lc = lin[:, c * 32:(c + 1) * 32]
        m = jnp.abs(lc) > babs
        babs = jnp.where(m, jnp.abs(lc), babs)
        bidx = jnp.where(m, c, bidx)
        bval = jnp.where(m, lc, bval)
    hsh = bidx + 4 * (bval < 0).astype(jnp.int32)
    qsc_ref[...] = q * SCALING
    vp_ref[...] = v
    invs_ref[...] = 1.0 / (norm * SCALING)
    hash_ref[...] = hsh


def _stage_a(xq, xv, wqt, bq2, wvt, bv2, wh):
    return pl.pallas_call(
        _proj_hash_body,
        grid=(T // _TBA,),
        in_specs=[
            pl.BlockSpec((_TBA, E), lambda i: (i, 0)),
            pl.BlockSpec((_TBA, E), lambda i: (i, 0)),
            pl.BlockSpec((E, E), lambda i: (0, 0)),
            pl.BlockSpec((1, E), lambda i: (0, 0)),
            pl.BlockSpec((E, E), lambda i: (0, 0)),
            pl.BlockSpec((1, E), lambda i: (0, 0)),
            pl.BlockSpec((E, 4 * NP), lambda i: (0, 0)),
        ],
        out_specs=[
            pl.BlockSpec((_TBA, E), lambda i: (i, 0)),
            pl.BlockSpec((_TBA, E), lambda i: (i, 0)),
            pl.BlockSpec((_TBA, 1), lambda i: (i, 0)),
            pl.BlockSpec((_TBA, NP), lambda i: (i, 0)),
        ],
        out_shape=[
            jax.ShapeDtypeStruct((T, E), _f32),
            jax.ShapeDtypeStruct((T, E), _f32),
            jax.ShapeDtypeStruct((T, 1), _f32),
            jax.ShapeDtypeStruct((T, NP), jnp.int32),
        ],
        interpret=_INTERPRET,
    )(xq, xv, wqt, bq2, wvt, bv2, wh)


_GC = 128
_NGC = T // _GC
_NV = T // 16


def _sc_wid():
    return lax.axis_index("s") * 2 + lax.axis_index("c")


def _sort_gather_body(hash_hbm, invs_hbm, qv_hbm,
                      sidx_hbm, shash_hbm, sinvs_hbm, sqv_hbm,
                      hv, invs_v, sidx_v, shash_v, sinvs_v, idx2, rowbuf,
                      nxt, sem):
    wid = _sc_wid()
    hh = lax.rem(wid, H)
    pltpu.sync_copy(hash_hbm.at[wid], hv)
    pltpu.sync_copy(invs_hbm, invs_v)

    def h1(i, carry):
        hvec = hv[pl.ds(i * 16, 16)]
        return tuple(carry[b] + jnp.sum((hvec == b).astype(jnp.int32))
                     for b in range(NH))

    c8 = lax.fori_loop(0, _NV, h1, (jnp.int32(0),) * NH)
    off = jnp.int32(0)
    for b in range(NH):
        nxt[b] = off
        off = off + c8[b]

    def h2(i, carry):
        hvec = hv[pl.ds(i * 16, 16)]
        pos = jnp.zeros((16,), jnp.int32)
        for b in range(NH):
            eq = hvec == b
            eqi = eq.astype(jnp.int32)
            cs = plsc.cumsum(eqi)
            nb = nxt[b]
            nbv = jnp.full((16,), nb, jnp.int32)
            pos = jnp.where(eq, nbv + cs - 1, pos)
            nxt[b] = nb + jnp.sum(eqi)
        tok = jnp.full((16,), i * 16, jnp.int32) + lax.iota(jnp.int32, 16)
        plsc.store_scatter(sidx_v, [pos], tok)
        plsc.store_scatter(shash_v, [pos], hvec)
        return carry

    lax.fori_loop(0, _NV, h2, 0)

    for c in range(_NGC):
        def h3(j, carry):
            base = c * _GC + j * 16
            iv = sidx_v[pl.ds(base, 16)]
            sinvs_v[pl.ds(base, 16)] = plsc.load_gather(invs_v, [iv])
            idx2[c, pl.ds(j * 16, 16)] = iv * H + jnp.full((16,), hh,
                                                          jnp.int32)
            return carry
        lax.fori_loop(0, _GC // 16, h3, 0)

    pltpu.sync_copy(sidx_v, sidx_hbm.at[wid])
    pltpu.sync_copy(shash_v, shash_hbm.at[wid])
    pltpu.sync_copy(sinvs_v, sinvs_hbm.at[wid])

    def gq(c, carry):
        pltpu.async_copy(qv_hbm.at[idx2.at[c]], rowbuf, sem).wait()
        pltpu.sync_copy(rowbuf, sqv_hbm.at[wid, pl.ds(c * _GC, _GC)])
        return carry

    lax.fori_loop(0, _NGC, gq, 0)


def _stage_b(hashes_t, invs_flat, qv):
    mesh = plsc.VectorSubcoreMesh(core_axis_name="c", subcore_axis_name="s")
    f = functools.partial(
        pl.kernel, mesh=mesh,
        out_type=[
            jax.ShapeDtypeStruct((NP, T), jnp.int32),
            jax.ShapeDtypeStruct((NP, T), jnp.int32),
            jax.ShapeDtypeStruct((NP, T), _f32),
            jax.ShapeDtypeStruct((NP, T, 2 * Dh), _f32),
        ],
        scratch_types=[
            pltpu.VMEM((T,), jnp.int32),
            pltpu.VMEM((T,), _f32),
            pltpu.VMEM((T,), jnp.int32),
            pltpu.VMEM((T,), jnp.int32),
            pltpu.VMEM((T,), _f32),
            pltpu.VMEM((_NGC, _GC), jnp.int32),
            pltpu.VMEM((_GC, 2 * Dh), _f32),
            pltpu.SMEM((16,), jnp.int32),
            pltpu.SemaphoreType.DMA,
        ],
        compiler_params=pltpu.CompilerParams(needs_layout_passes=False),
    )(_sort_gather_body)
    return f(hashes_t, invs_flat, qv)


def _unsort_body(outp_hbm, sidx_hbm, outu_hbm,
                 sidx_v, idx3, rowbuf, sem):
    wid = _sc_wid()
    pltpu.sync_copy(sidx_hbm.at[wid], sidx_v)

    for c in range(_NGC):
        def f1(j, carry):
            base = c * _GC + j * 16
            iv = sidx_v[pl.ds(base, 16)]
            idx3[c, pl.ds(j * 16, 16)] = iv * NP + jnp.full((16,), wid,
                                                            jnp.int32)
            return carry
        lax.fori_loop(0, _GC // 16, f1, 0)

    def f2(c, carry):
        pltpu.sync_copy(outp_hbm.at[wid, pl.ds(c * _GC, _GC)], rowbuf)
        pltpu.async_copy(rowbuf, outu_hbm.at[idx3.at[c]], sem).wait()
        return carry

    lax.fori_loop(0, _NGC, f2, 0)


def _stage_c(outp, sidx):
    mesh = plsc.VectorSubcoreMesh(core_axis_name="c", subcore_axis_name="s")
    f = functools.partial(
        pl.kernel, mesh=mesh,
        out_type=[
            jax.ShapeDtypeStruct((T * NP, 2 * Dh), _f32),
        ],
        scratch_types=[
            pltpu.VMEM((T,), jnp.int32),
            pltpu.VMEM((_NGC, _GC), jnp.int32),
            pltpu.VMEM((_GC, 2 * Dh), _f32),
            pltpu.SemaphoreType.DMA,
        ],
        compiler_params=pltpu.CompilerParams(needs_layout_passes=False),
    )(_unsort_body)
    return f(outp, sidx)[0]


def _attn_body(sqv_ref, sinvc_ref, shc_ref, spc_ref, shr_ref,
               spr_ref, out_ref):
    sqv = sqv_ref[0]
    sq = sqv[:, 0:Dh]
    sv = sqv[:, Dh:2 * Dh]
    ks = sq * sinvc_ref[0]
    hcol = shc_ref[0]
    pcol = spc_ref[0]
    hrow = shr_ref[0]
    prow = spr_ref[0]

    NB = 4
    QW = NB * CS
    KW = (NB + 2) * CS
    qc = 1 + lax.broadcasted_iota(jnp.int32, (QW, KW), 0) // CS
    kc = lax.broadcasted_iota(jnp.int32, (QW, KW), 1) // CS
    band_bad = jnp.abs(qc - kc) > 1

    def win(x, a, axis):
        lo = a - CS
        hi = a + (NB + 1) * CS
        if lo < 0:
            sl = [x[T + lo:T, :], x[0:hi, :]] if axis == 0 else \
                 [x[:, T + lo:T], x[:, 0:hi]]
            return jnp.concatenate(sl, axis=axis)
        if hi > T:
            sl = [x[lo:T, :], x[0:hi - T, :]] if axis == 0 else \
                 [x[:, lo:T], x[:, 0:hi - T]]
            return jnp.concatenate(sl, axis=axis)
        return x[lo:hi, :] if axis == 0 else x[:, lo:hi]

    for g in range(NC // NB):
        a = g * QW
        qn = sq[a:a + QW, :]
        hq = hcol[a:a + QW, :]
        pq = pcol[a:a + QW, :]
        kw = win(ks, a, 0)
        vw = win(sv, a, 0)
        hw = win(hrow, a, 1)
        pw = win(prow, a, 1)
        s = lax.dot_general(qn.astype(_bf16), kw.astype(_bf16),
                            (((1,), (1,)), ((), ())),
                            preferred_element_type=_f32)
        s = jnp.where(jnp.logical_or(band_bad, hq != hw), -1e9, s)
        s = jnp.where(pq == pw, -1e5, s)
        m = jnp.max(s, axis=1, keepdims=True)
        e = jnp.exp(s - m)
        ssum = jnp.sum(e, axis=1, keepdims=True)
        lse = m + jnp.log(ssum)
        probs = e / ssum
        o = lax.dot_general(probs.astype(_bf16), vw.astype(_bf16),
                            (((1,), (0,)), ((), ())),
                            preferred_element_type=_f32)
        out_ref[0, a:a + QW, 0:Dh] = o
        out_ref[0, a:a + QW, Dh:Dh + 1] = lse


def _stage_e(sqv, sinvc, shc, spc, shr, spr):
    return pl.pallas_call(
        _attn_body,
        grid=(NP,),
        in_specs=[
            pl.BlockSpec((1, T, 2 * Dh), lambda i: (i, 0, 0)),
            pl.BlockSpec((1, T, 1), lambda i: (i, 0, 0)),
            pl.BlockSpec((1, T, 1), lambda i: (i, 0, 0)),
            pl.BlockSpec((1, T, 1), lambda i: (i, 0, 0)),
            pl.BlockSpec((1, 1, T), lambda i: (i, 0, 0)),
            pl.BlockSpec((1, 1, T), lambda i: (i, 0, 0)),
        ],
        out_specs=pl.BlockSpec((1, T, 2 * Dh), lambda i: (i, 0, 0)),
        out_shape=jax.ShapeDtypeStruct((NP, T, 2 * Dh), _f32),
        interpret=_INTERPRET,
    )(sqv, sinvc, shc, spc, shr, spr)


_TBD = 512


def _combine_body(ou_ref, wot_ref, bo_ref, out_ref):
    pieces = []
    for h in range(H):
        c0 = h * 2 * Dh
        c1 = (H + h) * 2 * Dh
        o0 = ou_ref[:, c0:c0 + Dh]
        l0 = ou_ref[:, c0 + Dh:c0 + Dh + 1]
        o1 = ou_ref[:, c1:c1 + Dh]
        l1 = ou_ref[:, c1 + Dh:c1 + Dh + 1]
        d = l0 - l1
        w0 = jax.nn.sigmoid(d)
        w1 = jax.nn.sigmoid(-d)
        pieces.append(o0 * w0 + o1 * w1)
    comb = jnp.concatenate(pieces, axis=1)
    out_ref[...] = _dot_bf16(comb, wot_ref[...]) + bo_ref[...]


def _stage_d(outu, wot, bo2):
    return pl.pallas_call(
        _combine_body,
        grid=(T // _TBD,),
        in_specs=[
            pl.BlockSpec((_TBD, NP * 2 * Dh), lambda i: (i, 0)),
            pl.BlockSpec((E, E), lambda i: (0, 0)),
            pl.BlockSpec((1, E), lambda i: (0, 0)),
        ],
        out_specs=pl.BlockSpec((_TBD, E), lambda i: (i, 0)),
        out_shape=jax.ShapeDtypeStruct((T, E), _f32),
        interpret=_INTERPRET,
    )(outu, wot, bo2)


def kernel(query, key, value, Wq, bq, Wk, bk, Wv, bv, Wo, bo, hash_w):
    xq = query.reshape(T, E)
    xv = value.reshape(T, E)
    wqt = Wq.T
    wvt = Wv.T
    wot = Wo.T
    bq2 = bq.reshape(1, E)
    bv2 = bv.reshape(1, E)
    bo2 = bo.reshape(1, E)
    wh = jnp.einsum('rhfc,hg->hfcrg', hash_w,
                    jnp.eye(H, dtype=_f32)).reshape(E, 4 * NP)

    qsc, vp, invs, hashes = _stage_a(xq, xv, wqt, bq2, wvt, bv2, wh)

    hashes_t = hashes.T
    qv = jnp.concatenate([qsc.reshape(T, H, Dh), vp.reshape(T, H, Dh)],
                         axis=-1).reshape(T * H, 2 * Dh)
    sidx, shash, sinvs, sqv = _stage_b(hashes_t, invs.reshape(T), qv)

    outp = _stage_e(sqv, sinvs[:, :, None], shash[:, :, None],
                    sidx[:, :, None], shash[:, None, :], sidx[:, None, :])

    outu_flat = _stage_c(outp, sidx)
    outu = outu_flat.reshape(T, NP * 2 * Dh)

    out = _stage_d(outu, wot, bo2)
    return out.reshape(T, B, E)


# speedup vs baseline: 8.7106x; 1.0853x over previous
import functools

import jax
import jax.numpy as jnp
from jax import lax
from jax.experimental import pallas as pl
from jax.experimental.pallas import tpu as pltpu
from jax.experimental.pallas import tpu_sc as plsc

T, B, E, H = 4096, 1, 1024, 16
R, NH, CS = 2, 8, 64
Dh = E // H
SCALING = Dh ** -0.5
NC = T // CS
NP = R * H

_INTERPRET = False

_bf16 = jnp.bfloat16
_f32 = jnp.float32


def _dot_bf16(a, b):
    return lax.dot(a.astype(_bf16), b.astype(_bf16),
                   preferred_element_type=_f32)


_TBA = 512


def _proj_hash_body(xq_ref, xv_ref, wqt_ref, bq_ref, wvt_ref, bv_ref,
                    wh_ref, qv_ref, invs_ref, hash_ref):
    q = _dot_bf16(xq_ref[...], wqt_ref[...]) + bq_ref[...]
    norm = jnp.sqrt(jnp.sum(q * q, axis=1, keepdims=True))
    kh = q / norm
    v = _dot_bf16(xv_ref[...], wvt_ref[...]) + bv_ref[...]
    lin = _dot_bf16(kh, wh_ref[...])
    l0 = lin[:, 0:32]
    babs = jnp.abs(l0)
    bidx = jnp.zeros(l0.shape, jnp.int32)
    bval = l0
    for c in range(1, 4):
        lc = lin[:, c * 32:(c + 1) * 32]
        m = jnp.abs(lc) > babs
        babs = jnp.where(m, jnp.abs(lc), babs)
        bidx = jnp.where(m, c, bidx)
        bval = jnp.where(m, lc, bval)
    hsh = bidx + 4 * (bval < 0).astype(jnp.int32)
    qsc = q * SCALING
    for h in range(H):
        qv_ref[h, :, 0:Dh] = qsc[:, h * Dh:(h + 1) * Dh]
        qv_ref[h, :, Dh:2 * Dh] = v[:, h * Dh:(h + 1) * Dh]
    invs_ref[...] = 1.0 / (norm * SCALING)
    hash_ref[...] = jnp.transpose(hsh)


def _stage_a(xq, xv, wqt, bq2, wvt, bv2, wh):
    return pl.pallas_call(
        _proj_hash_body,
        grid=(T // _TBA,),
        in_specs=[
            pl.BlockSpec((_TBA, E), lambda i: (i, 0)),
            pl.BlockSpec((_TBA, E), lambda i: (i, 0)),
            pl.BlockSpec((E, E), lambda i: (0, 0)),
            pl.BlockSpec((1, E), lambda i: (0, 0)),
            pl.BlockSpec((E, E), lambda i: (0, 0)),
            pl.BlockSpec((1, E), lambda i: (0, 0)),
            pl.BlockSpec((E, 4 * NP), lambda i: (0, 0)),
        ],
        out_specs=[
            pl.BlockSpec((H, _TBA, 2 * Dh), lambda i: (0, i, 0)),
            pl.BlockSpec((_TBA, 1), lambda i: (i, 0)),
            pl.BlockSpec((NP, _TBA), lambda i: (0, i)),
        ],
        out_shape=[
            jax.ShapeDtypeStruct((H, T, 2 * Dh), _f32),
            jax.ShapeDtypeStruct((T, 1), _f32),
            jax.ShapeDtypeStruct((NP, T), jnp.int32),
        ],
        interpret=_INTERPRET,
    )(xq, xv, wqt, bq2, wvt, bv2, wh)


_GC = 128
_NGC = T // _GC
_NV = T // 16


def _sc_wid():
    return lax.axis_index("s") * 2 + lax.axis_index("c")


def _sort_gather_body(hash_hbm, invs_hbm, qv_hbm,
                      sidx_hbm, shash_hbm, sinvs_hbm, sqv_hbm,
                      hv, invs_v, sidx_v, shash_v, sinvs_v, idx2, rowbuf,
                      nxt, sem):
    wid = _sc_wid()
    hh = lax.rem(wid, H)
    pltpu.sync_copy(hash_hbm.at[wid], hv)
    pltpu.sync_copy(invs_hbm, invs_v)

    def h1(i, carry):
        hvec = hv[pl.ds(i * 16, 16)]
        return tuple(carry[b] + jnp.sum((hvec == b).astype(jnp.int32))
                     for b in range(NH))

    c8 = lax.fori_loop(0, _NV, h1, (jnp.int32(0),) * NH)
    off = jnp.int32(0)
    for b in range(NH):
        nxt[b] = off
        off = off + c8[b]

    def h2(i, carry):
        hvec = hv[pl.ds(i * 16, 16)]
        pos = jnp.zeros((16,), jnp.int32)
        for b in range(NH):
            eq = hvec == b
            eqi = eq.astype(jnp.int32)
            cs = plsc.cumsum(eqi)
            nb = nxt[b]
            nbv = jnp.full((16,), nb, jnp.int32)
            pos = jnp.where(eq, nbv + cs - 1, pos)
            nxt[b] = nb + jnp.sum(eqi)
        tok = jnp.full((16,), i * 16, jnp.int32) + lax.iota(jnp.int32, 16)
        plsc.store_scatter(sidx_v, [pos], tok)
        plsc.store_scatter(shash_v, [pos], hvec)
        return carry

    lax.fori_loop(0, _NV, h2, 0)

    for c in range(_NGC):
        def h3(j, carry):
            base = c * _GC + j * 16
            iv = sidx_v[pl.ds(base, 16)]
            sinvs_v[pl.ds(base, 16)] = plsc.load_gather(invs_v, [iv])
            idx2[c, pl.ds(j * 16, 16)] = iv + jnp.full((16,), hh * T,
                                                       jnp.int32)
            return carry
        lax.fori_loop(0, _GC // 16, h3, 0)

    pltpu.sync_copy(sidx_v, sidx_hbm.at[wid])
    pltpu.sync_copy(shash_v, shash_hbm.at[wid])
    pltpu.sync_copy(sinvs_v, sinvs_hbm.at[wid])

    def gq(c, carry):
        pltpu.async_copy(qv_hbm.at[idx2.at[c]], rowbuf, sem).wait()
        pltpu.sync_copy(rowbuf, sqv_hbm.at[wid, pl.ds(c * _GC, _GC)])
        return carry

    lax.fori_loop(0, _NGC, gq, 0)


def _stage_b(hashes_t, invs_flat, qv):
    mesh = plsc.VectorSubcoreMesh(core_axis_name="c", subcore_axis_name="s")
    f = functools.partial(
        pl.kernel, mesh=mesh,
        out_type=[
            jax.ShapeDtypeStruct((NP, T), jnp.int32),
            jax.ShapeDtypeStruct((NP, T), jnp.int32),
            jax.ShapeDtypeStruct((NP, T), _f32),
            jax.ShapeDtypeStruct((NP, T, 2 * Dh), _f32),
        ],
        scratch_types=[
            pltpu.VMEM((T,), jnp.int32),
            pltpu.VMEM((T,), _f32),
            pltpu.VMEM((T,), jnp.int32),
            pltpu.VMEM((T,), jnp.int32),
            pltpu.VMEM((T,), _f32),
            pltpu.VMEM((_NGC, _GC), jnp.int32),
            pltpu.VMEM((_GC, 2 * Dh), _f32),
            pltpu.SMEM((16,), jnp.int32),
            pltpu.SemaphoreType.DMA,
        ],
        compiler_params=pltpu.CompilerParams(needs_layout_passes=False),
    )(_sort_gather_body)
    return f(hashes_t, invs_flat, qv)


def _unsort_body(outp_hbm, sidx_hbm, outu_hbm,
                 sidx_v, idx3, rowbuf, sem):
    wid = _sc_wid()
    pltpu.sync_copy(sidx_hbm.at[wid], sidx_v)

    for c in range(_NGC):
        def f1(j, carry):
            base = c * _GC + j * 16
            iv = sidx_v[pl.ds(base, 16)]
            idx3[c, pl.ds(j * 16, 16)] = iv * NP + jnp.full((16,), wid,
                                                            jnp.int32)
            return carry
        lax.fori_loop(0, _GC // 16, f1, 0)

    def f2(c, carry):
        pltpu.sync_copy(outp_hbm.at[wid, pl.ds(c * _GC, _GC)], rowbuf)
        pltpu.async_copy(rowbuf, outu_hbm.at[idx3.at[c]], sem).wait()
        return carry

    lax.fori_loop(0, _NGC, f2, 0)


def _stage_c(outp, sidx):
    mesh = plsc.VectorSubcoreMesh(core_axis_name="c", subcore_axis_name="s")
    f = functools.partial(
        pl.kernel, mesh=mesh,
        out_type=[
            jax.ShapeDtypeStruct((T * NP, 2 * Dh), _f32),
        ],
        scratch_types=[
            pltpu.VMEM((T,), jnp.int32),
            pltpu.VMEM((_NGC, _GC), jnp.int32),
            pltpu.VMEM((_GC, 2 * Dh), _f32),
            pltpu.SemaphoreType.DMA,
        ],
        compiler_params=pltpu.CompilerParams(needs_layout_passes=False),
    )(_unsort_body)
    return f(outp, sidx)[0]


def _attn_body(sqv_ref, sinvc_ref, shc_ref, spc_ref, shr_ref,
               spr_ref, out_ref):

    NB = 4
    QW = NB * CS
    KW = (NB + 2) * CS
    qc = 1 + lax.broadcasted_iota(jnp.int32, (QW, KW), 0) // CS
    kc = lax.broadcasted_iota(jnp.int32, (QW, KW), 1) // CS
    band_bad = jnp.abs(qc - kc) > 1

    def win(x, a, axis):
        lo = a - CS
        hi = a + (NB + 1) * CS
        if lo < 0:
            sl = [x[T + lo:T, :], x[0:hi, :]] if axis == 0 else \
                 [x[:, T + lo:T], x[:, 0:hi]]
            return jnp.concatenate(sl, axis=axis)
        if hi > T:
            sl = [x[lo:T, :], x[0:hi - T, :]] if axis == 0 else \
                 [x[:, lo:T], x[:, 0:hi - T]]
            return jnp.concatenate(sl, axis=axis)
        return x[lo:hi, :] if axis == 0 else x[:, lo:hi]

    per = []
    for p in range(_PP):
        sqv = sqv_ref[p]
        sq = sqv[:, 0:Dh]
        sv = sqv[:, Dh:2 * Dh]
        per.append((sq, sv, sq * sinvc_ref[p], shc_ref[p], spc_ref[p],
                    shr_ref[p], spr_ref[p]))

    for g in range(NC // NB):
        a = g * QW
        for p in range(_PP):
            sq, sv, ks, hcol, pcol, hrow, prow = per[p]
            qn = sq[a:a + QW, :]
            hq = hcol[a:a + QW, :]
            pq = pcol[a:a + QW, :]
            kw = win(ks, a, 0)
            vw = win(sv, a, 0)
            hw = win(hrow, a, 1)
            pw = win(prow, a, 1)
            s = lax.dot_general(qn.astype(_bf16), kw.astype(_bf16),
                                (((1,), (1,)), ((), ())),
                                preferred_element_type=_f32)
            s = jnp.where(jnp.logical_or(band_bad, hq != hw), -1e9, s)
            s = jnp.where(pq == pw, -1e5, s)
            m = jnp.max(s, axis=1, keepdims=True)
            e = jnp.exp(s - m)
            ssum = jnp.sum(e, axis=1, keepdims=True)
            lse = m + jnp.log(ssum)
            probs = e / ssum
            o = lax.dot_general(probs.astype(_bf16), vw.astype(_bf16),
                                (((1,), (0,)), ((), ())),
                                preferred_element_type=_f32)
            out_ref[p, a:a + QW, 0:Dh] = o
            out_ref[p, a:a + QW, Dh:Dh + 1] = lse


_PP = 1


def _stage_e(sqv, sinvc, shc, spc, shr, spr):
    return pl.pallas_call(
        _attn_body,
        grid=(NP // _PP,),
        in_specs=[
            pl.BlockSpec((_PP, T, 2 * Dh), lambda i: (i, 0, 0)),
            pl.BlockSpec((_PP, T, 1), lambda i: (i, 0, 0)),
            pl.BlockSpec((_PP, T, 1), lambda i: (i, 0, 0)),
            pl.BlockSpec((_PP, T, 1), lambda i: (i, 0, 0)),
            pl.BlockSpec((_PP, 1, T), lambda i: (i, 0, 0)),
            pl.BlockSpec((_PP, 1, T), lambda i: (i, 0, 0)),
        ],
        out_specs=pl.BlockSpec((_PP, T, 2 * Dh), lambda i: (i, 0, 0)),
        out_shape=jax.ShapeDtypeStruct((NP, T, 2 * Dh), _f32),
        interpret=_INTERPRET,
    )(sqv, sinvc, shc, spc, shr, spr)


_TBD = 512


def _combine_body(ou_ref, wot_ref, bo_ref, out_ref):
    pieces = []
    for h in range(H):
        c0 = h * 2 * Dh
        c1 = (H + h) * 2 * Dh
        o0 = ou_ref[:, c0:c0 + Dh]
        l0 = ou_ref[:, c0 + Dh:c0 + Dh + 1]
        o1 = ou_ref[:, c1:c1 + Dh]
        l1 = ou_ref[:, c1 + Dh:c1 + Dh + 1]
        d = l0 - l1
        w0 = jax.nn.sigmoid(d)
        w1 = jax.nn.sigmoid(-d)
        pieces.append(o0 * w0 + o1 * w1)
    comb = jnp.concatenate(pieces, axis=1)
    out_ref[...] = _dot_bf16(comb, wot_ref[...]) + bo_ref[...]


def _stage_d(outu, wot, bo2):
    return pl.pallas_call(
        _combine_body,
        grid=(T // _TBD,),
        in_specs=[
            pl.BlockSpec((_TBD, NP * 2 * Dh), lambda i: (i, 0)),
            pl.BlockSpec((E, E), lambda i: (0, 0)),
            pl.BlockSpec((1, E), lambda i: (0, 0)),
        ],
        out_specs=pl.BlockSpec((_TBD, E), lambda i: (i, 0)),
        out_shape=jax.ShapeDtypeStruct((T, E), _f32),
        interpret=_INTERPRET,
    )(outu, wot, bo2)


def kernel(query, key, value, Wq, bq, Wk, bk, Wv, bv, Wo, bo, hash_w):
    xq = query.reshape(T, E)
    xv = value.reshape(T, E)
    wqt = Wq.T
    wvt = Wv.T
    wot = Wo.T
    bq2 = bq.reshape(1, E)
    bv2 = bv.reshape(1, E)
    bo2 = bo.reshape(1, E)
    wh = jnp.einsum('rhfc,hg->hfcrg', hash_w,
                    jnp.eye(H, dtype=_f32)).reshape(E, 4 * NP)

    qv3, invs, hashes_t = _stage_a(xq, xv, wqt, bq2, wvt, bv2, wh)

    qv = qv3.reshape(H * T, 2 * Dh)
    sidx, shash, sinvs, sqv = _stage_b(hashes_t, invs.reshape(T), qv)

    outp = _stage_e(sqv, sinvs[:, :, None], shash[:, :, None],
                    sidx[:, :, None], shash[:, None, :], sidx[:, None, :])

    outu_flat = _stage_c(outp, sidx)
    outu = outu_flat.reshape(T, NP * 2 * Dh)

    out = _stage_d(outu, wot, bo2)
    return out.reshape(T, B, E)


# speedup vs baseline: 9.0324x; 1.0369x over previous
import functools

import jax
import jax.numpy as jnp
from jax import lax
from jax.experimental import pallas as pl
from jax.experimental.pallas import tpu as pltpu
from jax.experimental.pallas import tpu_sc as plsc

T, B, E, H = 4096, 1, 1024, 16
R, NH, CS = 2, 8, 64
Dh = E // H
SCALING = Dh ** -0.5
NC = T // CS
NP = R * H

_INTERPRET = False

_bf16 = jnp.bfloat16
_f32 = jnp.float32


def _dot_bf16(a, b):
    return lax.dot(a.astype(_bf16), b.astype(_bf16),
                   preferred_element_type=_f32)


_TBA = 512


def _proj_hash_body(xq_ref, xv_ref, wqt_ref, bq_ref, wvt_ref, bv_ref,
                    wh_ref, qv_ref, invs_ref, hash_ref):
    q = _dot_bf16(xq_ref[...], wqt_ref[...]) + bq_ref[...]
    norm = jnp.sqrt(jnp.sum(q * q, axis=1, keepdims=True))
    kh = q / norm
    v = _dot_bf16(xv_ref[...], wvt_ref[...]) + bv_ref[...]
    lin = _dot_bf16(kh, wh_ref[...])
    l0 = lin[:, 0:32]
    babs = jnp.abs(l0)
    bidx = jnp.zeros(l0.shape, jnp.int32)
    bval = l0
    for c in range(1, 4):
        lc = lin[:, c * 32:(c + 1) * 32]
        m = jnp.abs(lc) > babs
        babs = jnp.where(m, jnp.abs(lc), babs)
        bidx = jnp.where(m, c, bidx)
        bval = jnp.where(m, lc, bval)
    hsh = bidx + 4 * (bval < 0).astype(jnp.int32)
    qsc = q * SCALING
    for h in range(H):
        qv_ref[h, :, 0:Dh] = qsc[:, h * Dh:(h + 1) * Dh]
        qv_ref[h, :, Dh:2 * Dh] = v[:, h * Dh:(h + 1) * Dh]
    invs_ref[...] = 1.0 / (norm * SCALING)
    hash_ref[...] = jnp.transpose(hsh)


def _stage_a(xq, xv, wqt, bq2, wvt, bv2, wh):
    return pl.pallas_call(
        _proj_hash_body,
        grid=(T // _TBA,),
        in_specs=[
            pl.BlockSpec((_TBA, E), lambda i: (i, 0)),
            pl.BlockSpec((_TBA, E), lambda i: (i, 0)),
            pl.BlockSpec((E, E), lambda i: (0, 0)),
            pl.BlockSpec((1, E), lambda i: (0, 0)),
            pl.BlockSpec((E, E), lambda i: (0, 0)),
            pl.BlockSpec((1, E), lambda i: (0, 0)),
            pl.BlockSpec((E, 4 * NP), lambda i: (0, 0)),
        ],
        out_specs=[
            pl.BlockSpec((H, _TBA, 2 * Dh), lambda i: (0, i, 0)),
            pl.BlockSpec((_TBA, 1), lambda i: (i, 0)),
            pl.BlockSpec((NP, _TBA), lambda i: (0, i)),
        ],
        out_shape=[
            jax.ShapeDtypeStruct((H, T, 2 * Dh), _f32),
            jax.ShapeDtypeStruct((T, 1), _f32),
            jax.ShapeDtypeStruct((NP, T), jnp.int32),
        ],
        interpret=_INTERPRET,
    )(xq, xv, wqt, bq2, wvt, bv2, wh)


_GC = 128
_NGC = T // _GC
_NV = T // 16


def _sc_wid():
    return lax.axis_index("s") * 2 + lax.axis_index("c")


def _sort_gather_body(hash_hbm, invs_hbm, qv_hbm,
                      sidx_hbm, shash_hbm, sinvs_hbm, sqv_hbm,
                      hv, invs_v, sidx_v, shash_v, sinvs_v, idx2, rowbuf,
                      rowbuf2, nxt, sem, sem2):
    wid = _sc_wid()
    hh = lax.rem(wid, H)
    pltpu.sync_copy(hash_hbm.at[wid], hv)
    pltpu.sync_copy(invs_hbm, invs_v)

    def h1(i, carry):
        hvec = hv[pl.ds(i * 16, 16)]
        return tuple(carry[b] + jnp.sum((hvec == b).astype(jnp.int32))
                     for b in range(NH))

    c8 = lax.fori_loop(0, _NV, h1, (jnp.int32(0),) * NH)
    off = jnp.int32(0)
    for b in range(NH):
        nxt[b] = off
        off = off + c8[b]

    def h2(i, carry):
        hvec = hv[pl.ds(i * 16, 16)]
        pos = jnp.zeros((16,), jnp.int32)
        for b in range(NH):
            eq = hvec == b
            eqi = eq.astype(jnp.int32)
            cs = plsc.cumsum(eqi)
            nb = nxt[b]
            nbv = jnp.full((16,), nb, jnp.int32)
            pos = jnp.where(eq, nbv + cs - 1, pos)
            nxt[b] = nb + jnp.sum(eqi)
        tok = jnp.full((16,), i * 16, jnp.int32) + lax.iota(jnp.int32, 16)
        plsc.store_scatter(sidx_v, [pos], tok)
        plsc.store_scatter(shash_v, [pos], hvec)
        return carry

    lax.fori_loop(0, _NV, h2, 0)

    for c in range(_NGC):
        def h3(j, carry):
            base = c * _GC + j * 16
            iv = sidx_v[pl.ds(base, 16)]
            sinvs_v[pl.ds(base, 16)] = plsc.load_gather(invs_v, [iv])
            idx2[c, pl.ds(j * 16, 16)] = iv + jnp.full((16,), hh * T,
                                                       jnp.int32)
            return carry
        lax.fori_loop(0, _GC // 16, h3, 0)

    pltpu.sync_copy(sidx_v, sidx_hbm.at[wid])
    pltpu.sync_copy(shash_v, shash_hbm.at[wid])
    pltpu.sync_copy(sinvs_v, sinvs_hbm.at[wid])

    rowbufs = (rowbuf, rowbuf2)
    sems = (sem, sem2)
    last = None
    for c in range(_NGC):
        cp = pltpu.async_copy(qv_hbm.at[idx2.at[c]], rowbufs[c % 2],
                              sems[c % 2])
        if last is not None:
            pc, pcp = last
            pcp.wait()
            pltpu.sync_copy(rowbufs[pc % 2],
                            sqv_hbm.at[wid, pl.ds(pc * _GC, _GC)])
        last = (c, cp)
    pc, pcp = last
    pcp.wait()
    pltpu.sync_copy(rowbufs[pc % 2], sqv_hbm.at[wid, pl.ds(pc * _GC, _GC)])


def _stage_b(hashes_t, invs_flat, qv):
    mesh = plsc.VectorSubcoreMesh(core_axis_name="c", subcore_axis_name="s")
    f = functools.partial(
        pl.kernel, mesh=mesh,
        out_type=[
            jax.ShapeDtypeStruct((NP, T), jnp.int32),
            jax.ShapeDtypeStruct((NP, T), jnp.int32),
            jax.ShapeDtypeStruct((NP, T), _f32),
            jax.ShapeDtypeStruct((NP, T, 2 * Dh), _f32),
        ],
        scratch_types=[
            pltpu.VMEM((T,), jnp.int32),
            pltpu.VMEM((T,), _f32),
            pltpu.VMEM((T,), jnp.int32),
            pltpu.VMEM((T,), jnp.int32),
            pltpu.VMEM((T,), _f32),
            pltpu.VMEM((_NGC, _GC), jnp.int32),
            pltpu.VMEM((_GC, 2 * Dh), _f32),
            pltpu.VMEM((_GC, 2 * Dh), _f32),
            pltpu.SMEM((16,), jnp.int32),
            pltpu.SemaphoreType.DMA,
            pltpu.SemaphoreType.DMA,
        ],
        compiler_params=pltpu.CompilerParams(needs_layout_passes=False),
    )(_sort_gather_body)
    return f(hashes_t, invs_flat, qv)


def _unsort_body(outp_hbm, sidx_hbm, outu_hbm,
                 sidx_v, idx3, rowbuf, rowbuf2, sem, sem2):
    wid = _sc_wid()
    pltpu.sync_copy(sidx_hbm.at[wid], sidx_v)

    for c in range(_NGC):
        def f1(j, carry):
            base = c * _GC + j * 16
            iv = sidx_v[pl.ds(base, 16)]
            idx3[c, pl.ds(j * 16, 16)] = iv * NP + jnp.full((16,), wid,
                                                            jnp.int32)
            return carry
        lax.fori_loop(0, _GC // 16, f1, 0)

    rowbufs = (rowbuf, rowbuf2)
    sems = (sem, sem2)
    scps = [None, None]
    for c in range(_NGC):
        b = c % 2
        if scps[b] is not None:
            scps[b].wait()
        pltpu.sync_copy(outp_hbm.at[wid, pl.ds(c * _GC, _GC)], rowbufs[b])
        scps[b] = pltpu.async_copy(rowbufs[b], outu_hbm.at[idx3.at[c]],
                                   sems[b])
    for b in range(2):
        scps[b].wait()


def _stage_c(outp, sidx):
    mesh = plsc.VectorSubcoreMesh(core_axis_name="c", subcore_axis_name="s")
    f = functools.partial(
        pl.kernel, mesh=mesh,
        out_type=[
            jax.ShapeDtypeStruct((T * NP, 2 * Dh), _f32),
        ],
        scratch_types=[
            pltpu.VMEM((T,), jnp.int32),
            pltpu.VMEM((_NGC, _GC), jnp.int32),
            pltpu.VMEM((_GC, 2 * Dh), _f32),
            pltpu.VMEM((_GC, 2 * Dh), _f32),
            pltpu.SemaphoreType.DMA,
            pltpu.SemaphoreType.DMA,
        ],
        compiler_params=pltpu.CompilerParams(needs_layout_passes=False),
    )(_unsort_body)
    return f(outp, sidx)[0]


def _attn_body(sqv_ref, sinvc_ref, shc_ref, spc_ref, shr_ref,
               spr_ref, out_ref):

    NB = 4
    QW = NB * CS
    KW = (NB + 2) * CS
    qc = 1 + lax.broadcasted_iota(jnp.int32, (QW, KW), 0) // CS
    kc = lax.broadcasted_iota(jnp.int32, (QW, KW), 1) // CS
    band_bad = jnp.abs(qc - kc) > 1

    def win(x, a, axis):
        lo = a - CS
        hi = a + (NB + 1) * CS
        if lo < 0:
            sl = [x[T + lo:T, :], x[0:hi, :]] if axis == 0 else \
                 [x[:, T + lo:T], x[:, 0:hi]]
            return jnp.concatenate(sl, axis=axis)
        if hi > T:
            sl = [x[lo:T, :], x[0:hi - T, :]] if axis == 0 else \
                 [x[:, lo:T], x[:, 0:hi - T]]
            return jnp.concatenate(sl, axis=axis)
        return x[lo:hi, :] if axis == 0 else x[:, lo:hi]

    per = []
    for p in range(_PP):
        sqv = sqv_ref[p]
        sq = sqv[:, 0:Dh]
        sv = sqv[:, Dh:2 * Dh]
        per.append((sq, sv, sq * sinvc_ref[p], shc_ref[p], spc_ref[p],
                    shr_ref[p], spr_ref[p]))

    for g in range(NC // NB):
        a = g * QW
        for p in range(_PP):
            sq, sv, ks, hcol, pcol, hrow, prow = per[p]
            qn = sq[a:a + QW, :]
            hq = hcol[a:a + QW, :]
            pq = pcol[a:a + QW, :]
            kw = win(ks, a, 0)
            vw = win(sv, a, 0)
            hw = win(hrow, a, 1)
            pw = win(prow, a, 1)
            s = lax.dot_general(qn.astype(_bf16), kw.astype(_bf16),
                                (((1,), (1,)), ((), ())),
                                preferred_element_type=_f32)
            s = jnp.where(jnp.logical_or(band_bad, hq != hw), -1e9, s)
            s = jnp.where(pq == pw, -1e5, s)
            m = jnp.max(s, axis=1, keepdims=True)
            e = jnp.exp(s - m)
            ssum = jnp.sum(e, axis=1, keepdims=True)
            lse = m + jnp.log(ssum)
            probs = e / ssum
            o = lax.dot_general(probs.astype(_bf16), vw.astype(_bf16),
                                (((1,), (0,)), ((), ())),
                                preferred_element_type=_f32)
            out_ref[p, a:a + QW, 0:Dh] = o
            out_ref[p, a:a + QW, Dh:Dh + 1] = lse


_PP = 1


def _stage_e(sqv, sinvc, shc, spc, shr, spr):
    return pl.pallas_call(
        _attn_body,
        grid=(NP // _PP,),
        in_specs=[
            pl.BlockSpec((_PP, T, 2 * Dh), lambda i: (i, 0, 0)),
            pl.BlockSpec((_PP, T, 1), lambda i: (i, 0, 0)),
            pl.BlockSpec((_PP, T, 1), lambda i: (i, 0, 0)),
            pl.BlockSpec((_PP, T, 1), lambda i: (i, 0, 0)),
            pl.BlockSpec((_PP, 1, T), lambda i: (i, 0, 0)),
            pl.BlockSpec((_PP, 1, T), lambda i: (i, 0, 0)),
        ],
        out_specs=pl.BlockSpec((_PP, T, 2 * Dh), lambda i: (i, 0, 0)),
        out_shape=jax.ShapeDtypeStruct((NP, T, 2 * Dh), _f32),
        interpret=_INTERPRET,
    )(sqv, sinvc, shc, spc, shr, spr)


_TBD = 512


def _combine_body(ou_ref, wot_ref, bo_ref, out_ref):
    pieces = []
    for h in range(H):
        c0 = h * 2 * Dh
        c1 = (H + h) * 2 * Dh
        o0 = ou_ref[:, c0:c0 + Dh]
        l0 = ou_ref[:, c0 + Dh:c0 + Dh + 1]
        o1 = ou_ref[:, c1:c1 + Dh]
        l1 = ou_ref[:, c1 + Dh:c1 + Dh + 1]
        d = l0 - l1
        w0 = jax.nn.sigmoid(d)
        w1 = jax.nn.sigmoid(-d)
        pieces.append(o0 * w0 + o1 * w1)
    comb = jnp.concatenate(pieces, axis=1)
    out_ref[...] = _dot_bf16(comb, wot_ref[...]) + bo_ref[...]


def _stage_d(outu, wot, bo2):
    return pl.pallas_call(
        _combine_body,
        grid=(T // _TBD,),
        in_specs=[
            pl.BlockSpec((_TBD, NP * 2 * Dh), lambda i: (i, 0)),
            pl.BlockSpec((E, E), lambda i: (0, 0)),
            pl.BlockSpec((1, E), lambda i: (0, 0)),
        ],
        out_specs=pl.BlockSpec((_TBD, E), lambda i: (i, 0)),
        out_shape=jax.ShapeDtypeStruct((T, E), _f32),
        interpret=_INTERPRET,
    )(outu, wot, bo2)


def kernel(query, key, value, Wq, bq, Wk, bk, Wv, bv, Wo, bo, hash_w):
    xq = query.reshape(T, E)
    xv = value.reshape(T, E)
    wqt = Wq.T
    wvt = Wv.T
    wot = Wo.T
    bq2 = bq.reshape(1, E)
    bv2 = bv.reshape(1, E)
    bo2 = bo.reshape(1, E)
    wh = jnp.einsum('rhfc,hg->hfcrg', hash_w,
                    jnp.eye(H, dtype=_f32)).reshape(E, 4 * NP)

    qv3, invs, hashes_t = _stage_a(xq, xv, wqt, bq2, wvt, bv2, wh)

    qv = qv3.reshape(H * T, 2 * Dh)
    sidx, shash, sinvs, sqv = _stage_b(hashes_t, invs.reshape(T), qv)

    outp = _stage_e(sqv, sinvs[:, :, None], shash[:, :, None],
                    sidx[:, :, None], shash[:, None, :], sidx[:, None, :])

    outu_flat = _stage_c(outp, sidx)
    outu = outu_flat.reshape(T, NP * 2 * Dh)

    out = _stage_d(outu, wot, bo2)
    return out.reshape(T, B, E)
